# baseline (speedup 1.0000x reference)
import functools

import jax
import jax.numpy as jnp
from jax import lax
from jax.experimental import pallas as pl
from jax.experimental.pallas import tpu as pltpu
from jax.experimental.pallas import tpu_sc as plsc

NC = 2
NS = 16
L = 16

NPAD = 2048
CHUNK = 128


@functools.cache
def _unpool_kernel(B, M, N_in, K, C):
    assert M % L == 0 and K % L == 0
    assert B == 2 * NC
    per_core_rows = 2 * M
    n_chunks = per_core_rows // CHUNK
    tail = per_core_rows - n_chunks * CHUNK
    assert tail % 8 == 0
    per_tile = -(-n_chunks // NS)
    zero_row = B * N_in
    mesh = plsc.VectorSubcoreMesh(core_axis_name="c", subcore_axis_name="s")

    @functools.partial(
        pl.kernel,
        out_type=jax.ShapeDtypeStruct((B * M, C), jnp.float32),
        mesh=mesh,
        scratch_types=[
            pltpu.VMEM((M,), jnp.int32),
            pltpu.VMEM((2, K), jnp.int32),
            pltpu.VMEM((M,), jnp.int32),
            pltpu.VMEM_SHARED((2 * M,), jnp.int32),
            pltpu.VMEM((CHUNK,), jnp.int32),
            pltpu.VMEM((CHUNK,), jnp.int32),
            pltpu.VMEM((CHUNK, C), jnp.float32),
            pltpu.VMEM((CHUNK, C), jnp.float32),
            pltpu.SemaphoreType.DMA,
            pltpu.SemaphoreType.DMA,
            pltpu.SemaphoreType.DMA,
            pltpu.SemaphoreType.DMA,
        ],
        compiler_params=pltpu.CompilerParams(needs_layout_passes=False),
    )
    def unpool(mask_hbm, order_hbm, table_hbm, out_hbm,
               mp_v, order_v, g_v, idx_sh,
               i0, i1, r0, r1, sg0, sg1, sw0, sw1):
        c = lax.axis_index("c")
        s = lax.axis_index("s")

        @pl.when(s < 2)
        def _():
            b = 2 * c + s
            pltpu.sync_copy(mask_hbm.at[b], mp_v)
            pltpu.sync_copy(order_hbm.at[b], order_v)
            boff = b * N_in
            iota = lax.iota(jnp.int32, L)

            def lane_bcast(v, j):
                return v.at[jnp.full((L,), j, jnp.int32)].get(
                    mode="promise_in_bounds"
                )

            def p1(i, carry):
                v = mp_v[pl.ds(i * L, L)]
                cs = plsc.cumsum(v)
                zspread = zero_row + ((iota + i * L) & (NPAD - 1))
                posz = jnp.where(v > 0, cs + carry + (boff - 1), zspread)
                mp_v[pl.ds(i * L, L)] = posz
                g_v[pl.ds(i * L, L)] = iota + i * L
                return carry + lane_bcast(cs, L - 1)

            lax.fori_loop(0, M // L, p1, jnp.zeros((L,), jnp.int32))

            lane0 = iota == 0
            rots = [jnp.where(iota < L - r, iota + r, iota + r - L)
                    for r in range(1, L)]

            def p2(i, _):
                base = K - (i + 1) * L
                fv = lax.rev(order_v[0, pl.ds(base, L)], (0,))
                tv = lax.rev(order_v[1, pl.ds(base, L)], (0,))
                conf = jnp.zeros((L,), jnp.bool_)
                for r in rots:
                    fr = fv.at[r].get(mode="promise_in_bounds")
                    tr = tv.at[r].get(mode="promise_in_bounds")
                    conf = conf | (tv == fr) | (tv == tr)

                def fast():
                    gf = plsc.load_gather(g_v, [fv])
                    plsc.store_scatter(g_v, [tv], gf)

                def slow():
                    for j in range(L):
                        fj = lane_bcast(fv, j)
                        tj = lane_bcast(tv, j)
                        gf = plsc.load_gather(g_v, [fj])
                        plsc.store_scatter(g_v, [tj], gf, mask=lane0)

                lax.cond(jnp.any(conf), slow, fast)
                return 0

            lax.fori_loop(0, K // L, p2, 0)

            def p3(i, _):
                gv = g_v[pl.ds(i * L, L)]
                g_v[pl.ds(i * L, L)] = plsc.load_gather(mp_v, [gv])
                return 0

            lax.fori_loop(0, M // L, p3, 0)
            pltpu.sync_copy(g_v, idx_sh.at[pl.ds(s * M, M)])

        plsc.subcore_barrier()

        idx_v = [i0, i1]
        rows_v = [r0, r1]
        sg = [sg0, sg1]
        sw = [sw0, sw1]
        row0 = c * per_core_rows

        def cid(j):
            return s + NS * j

        def gather_desc(sl):
            return pltpu.make_async_copy(table_hbm.at[idx_v[sl]], rows_v[sl],
                                         sg[sl])

        def wb_desc(sl, j):
            return pltpu.make_async_copy(
                rows_v[sl], out_hbm.at[pl.ds(row0 + cid(j) * CHUNK, CHUNK)],
                sw[sl],
            )

        for j in range(per_tile + 2):
            sl = j % 2
            if j >= 2:
                @pl.when(cid(j - 2) < n_chunks)
                def _(j=j, sl=sl):
                    wb_desc(sl, j - 2).wait()
            if j < per_tile:
                @pl.when(cid(j) < n_chunks)
                def _(j=j, sl=sl):
                    pltpu.sync_copy(idx_sh.at[pl.ds(cid(j) * CHUNK, CHUNK)],
                                    idx_v[sl])
                    gather_desc(sl).start()
            if 1 <= j <= per_tile:
                psl = (j - 1) % 2

                @pl.when(cid(j - 1) < n_chunks)
                def _(j=j, psl=psl):
                    gather_desc(psl).wait()
                    wb_desc(psl, j - 1).start()

        if tail:
            @pl.when(s == NS - 1)
            def _():
                t0 = n_chunks * CHUNK
                ti = i0.at[pl.ds(0, tail)]
                tr = r0.at[pl.ds(0, tail)]
                pltpu.sync_copy(idx_sh.at[pl.ds(t0, tail)], ti)
                pltpu.async_copy(table_hbm.at[ti], tr, sg0).wait()
                pltpu.sync_copy(tr, out_hbm.at[pl.ds(row0 + t0, tail)])

    return unpool


def kernel(images, mask, order):
    B, N_in, C = images.shape
    M = mask.shape[1]
    K = order.shape[2]

    table = jnp.concatenate(
        [images.reshape(B * N_in, C), jnp.zeros((NPAD, C), images.dtype)], axis=0
    )
    out = _unpool_kernel(B, M, N_in, K, C)(
        mask.astype(jnp.int32), order.astype(jnp.int32), table
    )
    return out.reshape(B, M, C)


# speedup vs baseline: 1.1483x; 1.1483x over previous
import functools

import jax
import jax.numpy as jnp
from jax import lax
from jax.experimental import pallas as pl
from jax.experimental.pallas import tpu as pltpu
from jax.experimental.pallas import tpu_sc as plsc

NC = 2
NS = 16
L = 16


def _widx():
    return lax.axis_index("s") * NC + lax.axis_index("c")


NPAD = 2048
HASH = 8192


@functools.cache
def _resolve_kernel(B, M, N_in, K):
    assert M % L == 0
    zero_row = B * N_in
    mesh = plsc.VectorSubcoreMesh(core_axis_name="c", subcore_axis_name="s")

    @functools.partial(
        pl.kernel,
        out_type=jax.ShapeDtypeStruct((B * M,), jnp.int32),
        mesh=mesh,
        scratch_types=[
            pltpu.VMEM((M,), jnp.int32),
            pltpu.VMEM((2, K), jnp.int32),
            pltpu.VMEM((M,), jnp.int32),
            pltpu.VMEM((M,), jnp.int32),
            pltpu.VMEM((HASH,), jnp.int32),
            pltpu.SemaphoreType.DMA,
        ],
        compiler_params=pltpu.CompilerParams(needs_layout_passes=False),
    )
    def resolve(mask_hbm, order_hbm, idx_hbm, mp_v, order_v, g_v, out_v,
                marks_v, sem_o):
        wid = _widx()

        @pl.when(wid < B)
        def _():
            b = wid
            order_dma = pltpu.async_copy(order_hbm.at[b], order_v, sem_o)
            pltpu.sync_copy(mask_hbm.at[b], mp_v)
            boff = b * N_in
            iota = lax.iota(jnp.int32, L)

            def lane_bcast(v, j):
                return v.at[jnp.full((L,), j, jnp.int32)].get(
                    mode="promise_in_bounds"
                )

            with jax.named_scope("p1_cumsum"):
                def p1(i, carry):
                    v = mp_v[pl.ds(i * L, L)]
                    cs = plsc.cumsum(v)
                    zspread = zero_row + ((iota + i * L) & (NPAD - 1))
                    posz = jnp.where(v > 0, cs + carry + (boff - 1), zspread)
                    mp_v[pl.ds(i * L, L)] = posz
                    g_v[pl.ds(i * L, L)] = iota + i * L
                    return carry + lane_bcast(cs, L - 1)

                lax.fori_loop(0, M // L, p1, jnp.zeros((L,), jnp.int32))

                @plsc.parallel_loop(0, HASH // L)
                def _(i):
                    marks_v[pl.ds(i * L, L)] = jnp.zeros((L,), jnp.int32)

            order_dma.wait()

            lane0 = iota == 0

            def p2(i, _):
                base = K - (i + 1) * L
                fv = lax.rev(order_v[0, pl.ds(base, L)], (0,))
                tv = lax.rev(order_v[1, pl.ds(base, L)], (0,))
                tagbase = (i + 1) * L
                ht = tv & (HASH - 1)
                plsc.store_scatter(marks_v, [ht], tagbase + iota)
                rt = plsc.load_gather(marks_v, [ht])
                rf = plsc.load_gather(marks_v, [fv & (HASH - 1)])
                conf = (rt != tagbase + iota) | (rf >= tagbase)

                def fast():
                    gf = plsc.load_gather(g_v, [fv])
                    plsc.store_scatter(g_v, [tv], gf)

                def slow():
                    for j in range(L):
                        fj = lane_bcast(fv, j)
                        tj = lane_bcast(tv, j)
                        gf = plsc.load_gather(g_v, [fj])
                        plsc.store_scatter(g_v, [tj], gf, mask=lane0)

                lax.cond(jnp.any(conf), slow, fast)
                return 0

            with jax.named_scope("p2_chain"):
                lax.fori_loop(0, K // L, p2, 0)

            with jax.named_scope("p3_compose"):
                @plsc.parallel_loop(0, M // L)
                def _(i):
                    gv = g_v[pl.ds(i * L, L)]
                    out_v[pl.ds(i * L, L)] = plsc.load_gather(mp_v, [gv])

            pltpu.sync_copy(out_v, idx_hbm.at[pl.ds(b * M, M)])

    return resolve


@functools.cache
def _gather_kernel(R, C, n_table):
    CHUNK = 128
    assert R % CHUNK == 0
    n_chunks = R // CHUNK
    n_tiles = NC * NS
    per_tile = -(-n_chunks // n_tiles)
    mesh = plsc.VectorSubcoreMesh(core_axis_name="c", subcore_axis_name="s")

    @functools.partial(
        pl.kernel,
        out_type=jax.ShapeDtypeStruct((R, C), jnp.float32),
        mesh=mesh,
        scratch_types=[
            pltpu.VMEM((CHUNK,), jnp.int32),
            pltpu.VMEM((CHUNK,), jnp.int32),
            pltpu.VMEM((CHUNK, C), jnp.float32),
            pltpu.VMEM((CHUNK, C), jnp.float32),
            pltpu.SemaphoreType.DMA,
            pltpu.SemaphoreType.DMA,
            pltpu.SemaphoreType.DMA,
            pltpu.SemaphoreType.DMA,
        ],
    )
    def gather(table_hbm, idx_hbm, out_hbm, i0, i1, r0, r1, sg0, sg1, sw0, sw1):
        wid = _widx()
        idx_v = [i0, i1]
        rows_v = [r0, r1]
        sg = [sg0, sg1]
        sw = [sw0, sw1]

        def cid(j):
            return wid + n_tiles * j

        def gather_desc(s):
            return pltpu.make_async_copy(table_hbm.at[idx_v[s]], rows_v[s], sg[s])

        def wb_desc(s, j):
            return pltpu.make_async_copy(
                rows_v[s], out_hbm.at[pl.ds(cid(j) * CHUNK, CHUNK)], sw[s]
            )

        for j in range(per_tile + 2):
            s = j % 2
            if j >= 2:
                @pl.when(cid(j - 2) < n_chunks)
                def _(j=j, s=s):
                    wb_desc(s, j - 2).wait()
            if j < per_tile:
                @pl.when(cid(j) < n_chunks)
                def _(j=j, s=s):
                    pltpu.sync_copy(idx_hbm.at[cid(j)], idx_v[s])
                    gather_desc(s).start()
            if 1 <= j <= per_tile:
                ps = (j - 1) % 2

                @pl.when(cid(j - 1) < n_chunks)
                def _(j=j, ps=ps):
                    gather_desc(ps).wait()
                    wb_desc(ps, j - 1).start()

    return gather


def kernel(images, mask, order):
    B, N_in, C = images.shape
    M = mask.shape[1]
    K = order.shape[2]

    idx = _resolve_kernel(B, M, N_in, K)(
        mask.astype(jnp.int32), order.astype(jnp.int32)
    )
    table = jnp.concatenate(
        [images.reshape(B * N_in, C), jnp.zeros((NPAD, C), images.dtype)], axis=0
    )
    out = _gather_kernel(B * M, C, B * N_in + NPAD)(
        table, idx.reshape(B * M // 128, 128)
    )
    return out.reshape(B, M, C)


# speedup vs baseline: 1.2814x; 1.1158x over previous
import functools

import jax
import jax.numpy as jnp
from jax import lax
from jax.experimental import pallas as pl
from jax.experimental.pallas import tpu as pltpu
from jax.experimental.pallas import tpu_sc as plsc

NC = 2
NS = 16
L = 16


def _widx():
    return lax.axis_index("s") * NC + lax.axis_index("c")


NPAD = 2048
HASH = 8192


@functools.cache
def _resolve_kernel(B, M, N_in, K, idx_pad):
    assert M % L == 0
    zero_row = B * N_in
    mesh = plsc.VectorSubcoreMesh(core_axis_name="c", subcore_axis_name="s")

    @functools.partial(
        pl.kernel,
        out_type=jax.ShapeDtypeStruct((idx_pad,), jnp.int32),
        mesh=mesh,
        scratch_types=[
            pltpu.VMEM((M,), jnp.int32),
            pltpu.VMEM((2, K), jnp.int32),
            pltpu.VMEM((M,), jnp.int32),
            pltpu.VMEM((M,), jnp.int32),
            pltpu.VMEM((HASH,), jnp.int32),
            pltpu.SemaphoreType.DMA,
        ],
        compiler_params=pltpu.CompilerParams(needs_layout_passes=False),
    )
    def resolve(mask_hbm, order_hbm, idx_hbm, mp_v, order_v, g_v, out_v,
                marks_v, sem_o):
        wid = _widx()

        @pl.when(wid < B)
        def _():
            b = wid
            order_dma = pltpu.async_copy(order_hbm.at[b], order_v, sem_o)
            pltpu.sync_copy(mask_hbm.at[b], mp_v)
            boff = b * N_in
            iota = lax.iota(jnp.int32, L)

            def lane_bcast(v, j):
                return v.at[jnp.full((L,), j, jnp.int32)].get(
                    mode="promise_in_bounds"
                )

            with jax.named_scope("p1_cumsum"):
                U = 5
                assert M % (U * L) == 0

                def p1(i, carry):
                    css = []
                    for u in range(U):
                        css.append(plsc.cumsum(mp_v[pl.ds((i * U + u) * L, L)]))
                    for u in range(U):
                        off = (i * U + u) * L
                        v = mp_v[pl.ds(off, L)]
                        zspread = zero_row + ((iota + off) & (NPAD - 1))
                        posz = jnp.where(v > 0, css[u] + carry + (boff - 1),
                                         zspread)
                        mp_v[pl.ds(off, L)] = posz
                        g_v[pl.ds(off, L)] = iota + off
                        carry = carry + lane_bcast(css[u], L - 1)
                    return carry

                lax.fori_loop(0, M // (U * L), p1, jnp.zeros((L,), jnp.int32))

                @plsc.parallel_loop(0, HASH // L, unroll=8)
                def _(i):
                    marks_v[pl.ds(i * L, L)] = jnp.zeros((L,), jnp.int32)

            order_dma.wait()

            lane0 = iota == 0

            def p2(i, _):
                base = K - (i + 1) * L
                fv = lax.rev(order_v[0, pl.ds(base, L)], (0,))
                tv = lax.rev(order_v[1, pl.ds(base, L)], (0,))
                tagbase = (i + 1) * L
                ht = tv & (HASH - 1)
                plsc.store_scatter(marks_v, [ht], tagbase + iota)
                rt = plsc.load_gather(marks_v, [ht])
                rf = plsc.load_gather(marks_v, [fv & (HASH - 1)])
                conf = (rt != tagbase + iota) | (rf >= tagbase)

                def fast():
                    gf = plsc.load_gather(g_v, [fv])
                    plsc.store_scatter(g_v, [tv], gf)

                def slow():
                    for j in range(L):
                        fj = lane_bcast(fv, j)
                        tj = lane_bcast(tv, j)
                        gf = plsc.load_gather(g_v, [fj])
                        plsc.store_scatter(g_v, [tj], gf, mask=lane0)

                lax.cond(jnp.any(conf), slow, fast)
                return 0

            with jax.named_scope("p2_chain"):
                lax.fori_loop(0, K // L, p2, 0)

            with jax.named_scope("p3_compose"):
                @plsc.parallel_loop(0, M // L, unroll=5)
                def _(i):
                    gv = g_v[pl.ds(i * L, L)]
                    out_v[pl.ds(i * L, L)] = plsc.load_gather(mp_v, [gv])

            pltpu.sync_copy(out_v, idx_hbm.at[pl.ds(b * M, M)])

    return resolve


NBUF = 3


@functools.cache
def _gather_kernel(R, C, n_table):
    CHUNK = 128
    assert R % CHUNK == 0
    n_chunks = R // CHUNK
    n_tiles = NC * NS
    per_tile = -(-n_chunks // n_tiles)
    n_big = n_chunks - n_tiles * (per_tile - 1)
    mesh = plsc.VectorSubcoreMesh(core_axis_name="c", subcore_axis_name="s")

    @functools.partial(
        pl.kernel,
        out_type=jax.ShapeDtypeStruct((R, C), jnp.float32),
        mesh=mesh,
        scratch_types=[
            pltpu.VMEM((per_tile * CHUNK,), jnp.int32),
            [pltpu.VMEM((CHUNK, C), jnp.float32) for _ in range(NBUF)],
            [pltpu.SemaphoreType.DMA for _ in range(NBUF)],
            [pltpu.SemaphoreType.DMA for _ in range(NBUF)],
        ],
    )
    def gather(table_hbm, idx_hbm, out_hbm, idx_all, rows_v, sg, sw):
        wid = _widx()
        base = jnp.where(wid < n_big, wid * per_tile,
                         n_big * per_tile + (wid - n_big) * (per_tile - 1))
        cnt = jnp.where(wid < n_big, per_tile, per_tile - 1)
        pltpu.sync_copy(idx_hbm.at[pl.ds(base * CHUNK, per_tile * CHUNK)],
                        idx_all)

        def gather_desc(sl, j):
            return pltpu.make_async_copy(
                table_hbm.at[idx_all.at[pl.ds(j * CHUNK, CHUNK)]],
                rows_v[sl], sg[sl])

        def wb_desc(sl, j):
            return pltpu.make_async_copy(
                rows_v[sl], out_hbm.at[pl.ds((base + j) * CHUNK, CHUNK)],
                sw[sl],
            )

        for j in range(per_tile + NBUF):
            sl = j % NBUF
            if j >= NBUF:
                @pl.when(j - NBUF < cnt)
                def _(j=j, sl=sl):
                    wb_desc(sl, j - NBUF).wait()
            if j < per_tile:
                @pl.when(j < cnt)
                def _(j=j, sl=sl):
                    gather_desc(sl, j).start()
            if 1 <= j <= per_tile:
                psl = (j - 1) % NBUF

                @pl.when(j - 1 < cnt)
                def _(j=j, psl=psl):
                    gather_desc(psl, j - 1).wait()
                    wb_desc(psl, j - 1).start()

    return gather


def kernel(images, mask, order):
    B, N_in, C = images.shape
    M = mask.shape[1]
    K = order.shape[2]

    n_chunks = B * M // 128
    n_tiles = NC * NS
    per_tile = -(-n_chunks // n_tiles)
    idx_pad = n_tiles * per_tile * 128

    idx = _resolve_kernel(B, M, N_in, K, idx_pad)(
        mask.astype(jnp.int32), order.astype(jnp.int32)
    )
    table = jnp.concatenate(
        [images.reshape(B * N_in, C), jnp.zeros((NPAD, C), images.dtype)], axis=0
    )
    out = _gather_kernel(B * M, C, B * N_in + NPAD)(table, idx)
    return out.reshape(B, M, C)
